# Initial kernel scaffold; baseline (speedup 1.0000x reference)
import jax
import jax.numpy as jnp
from jax.experimental import pallas as pl


def kernel(src_node_type, dst_node_type, embedding):
    raise NotImplementedError("write your pallas kernel here")


# speedup vs baseline: 2.9307x; 2.9307x over previous
import jax
import jax.numpy as jnp
from jax import lax
from jax.experimental import pallas as pl
from jax.experimental.pallas import tpu as pltpu
from jax.experimental.pallas import tpu_sc as plsc

_E = 320000
_DIM = 128
_NC = 2
_NS = 16
_NW = _NC * _NS
_BPW = _E // _NW
_CHUNK = 80
_NCHUNK = _BPW // _CHUNK
_L = 16


def _edge_embed_body(src_hbm, dst_hbm, emb_hbm, out_hbm,
                     src_v, dst_v, et_v, rows_v, sem):
    wid = lax.axis_index("s") * _NC + lax.axis_index("c")
    base = wid * _BPW

    pltpu.sync_copy(src_hbm.at[pl.ds(base, _BPW)], src_v)
    pltpu.sync_copy(dst_hbm.at[pl.ds(base, _BPW)], dst_v)

    def _compute(i, carry):
        x = src_v[pl.ds(i * _L, _L)]
        y = dst_v[pl.ds(i * _L, _L)]
        a = jnp.abs(x - y) - 1
        et_v[pl.ds(i * _L, _L)] = x * y + ((a * a) >> 2)
        return carry

    lax.fori_loop(0, _BPW // _L, _compute, 0)

    def _chunk(j, carry):
        cb = j * _CHUNK
        pltpu.async_copy(
            emb_hbm.at[et_v.at[pl.ds(cb, _CHUNK)]], rows_v, sem).wait()
        pltpu.sync_copy(rows_v, out_hbm.at[pl.ds(base + cb, _CHUNK)])
        return carry

    lax.fori_loop(0, _NCHUNK, _chunk, 0)


def kernel(src_node_type, dst_node_type, embedding):
    mesh = plsc.VectorSubcoreMesh(core_axis_name="c", subcore_axis_name="s")
    f = pl.kernel(
        _edge_embed_body,
        out_type=jax.ShapeDtypeStruct((_E, _DIM), jnp.float32),
        mesh=mesh,
        scratch_types=[
            pltpu.VMEM((_BPW,), jnp.int32),
            pltpu.VMEM((_BPW,), jnp.int32),
            pltpu.VMEM((_BPW,), jnp.int32),
            pltpu.VMEM((_CHUNK, _DIM), jnp.float32),
            pltpu.SemaphoreType.DMA,
        ],
    )
    return f(src_node_type.astype(jnp.int32),
             dst_node_type.astype(jnp.int32),
             embedding)


# speedup vs baseline: 3.8497x; 1.3136x over previous
import jax
import jax.numpy as jnp
from jax import lax
from jax.experimental import pallas as pl
from jax.experimental.pallas import tpu as pltpu
from jax.experimental.pallas import tpu_sc as plsc

_E = 320000
_DIM = 128
_NC = 2
_NS = 16
_NW = _NC * _NS
_BPW = _E // _NW
_CHUNK = 80
_NCHUNK = _BPW // _CHUNK
_NBUF = 5
_L = 16


def _edge_embed_body(src_hbm, dst_hbm, emb_hbm, out_hbm,
                     src_v, dst_v, et_v, rows_v, gsem, wsem):
    wid = lax.axis_index("s") * _NC + lax.axis_index("c")
    base = wid * _BPW

    pltpu.sync_copy(src_hbm.at[pl.ds(base, _BPW)], src_v)
    pltpu.sync_copy(dst_hbm.at[pl.ds(base, _BPW)], dst_v)

    def compute_chunk(c):
        for i in range(_CHUNK // _L):
            off = c * _CHUNK + i * _L
            x = src_v[pl.ds(off, _L)]
            y = dst_v[pl.ds(off, _L)]
            a = jnp.abs(x - y) - 1
            et_v[pl.ds(off, _L)] = x * y + ((a * a) >> 2)

    def gather_desc(c, b):
        return pltpu.make_async_copy(
            emb_hbm.at[et_v.at[pl.ds(c * _CHUNK, _CHUNK)]],
            rows_v.at[b], gsem.at[b])

    def wb_desc(c, b):
        return pltpu.make_async_copy(
            rows_v.at[b],
            out_hbm.at[pl.ds(base + c * _CHUNK, _CHUNK)], wsem.at[b])

    for c in range(_NBUF - 1):
        compute_chunk(c)
        gather_desc(c, c).start()
    gather_desc(0, 0).wait()
    wb_desc(0, 0).start()
    compute_chunk(_NBUF - 1)
    gather_desc(_NBUF - 1, _NBUF - 1).start()

    def block(blk, carry):
        for i in range(_NBUF):
            c = blk * _NBUF + 1 + i
            b = (1 + i) % _NBUF
            bp = i % _NBUF
            gather_desc(c, b).wait()
            wb_desc(c, b).start()
            wb_desc(c - 1, bp).wait()
            compute_chunk(c + _NBUF - 1)
            gather_desc(c + _NBUF - 1, bp).start()
        return carry

    lax.fori_loop(0, (_NCHUNK - _NBUF) // _NBUF, block, 0)

    for i in range(_NBUF - 1):
        c = _NCHUNK - _NBUF + 1 + i
        b = c % _NBUF
        gather_desc(c, b).wait()
        wb_desc(c, b).start()
    for b in range(_NBUF):
        wb_desc(_NCHUNK - _NBUF + b, b).wait()


def kernel(src_node_type, dst_node_type, embedding):
    mesh = plsc.VectorSubcoreMesh(core_axis_name="c", subcore_axis_name="s")
    f = pl.kernel(
        _edge_embed_body,
        out_type=jax.ShapeDtypeStruct((_E, _DIM), jnp.float32),
        mesh=mesh,
        scratch_types=[
            pltpu.VMEM((_BPW,), jnp.int32),
            pltpu.VMEM((_BPW,), jnp.int32),
            pltpu.VMEM((_BPW,), jnp.int32),
            pltpu.VMEM((_NBUF, _CHUNK, _DIM), jnp.float32),
            pltpu.SemaphoreType.DMA((_NBUF,)),
            pltpu.SemaphoreType.DMA((_NBUF,)),
        ],
    )
    return f(src_node_type.astype(jnp.int32),
             dst_node_type.astype(jnp.int32),
             embedding)


# speedup vs baseline: 8.6673x; 2.2514x over previous
import jax
import jax.numpy as jnp
from jax import lax
from jax.experimental import pallas as pl
from jax.experimental.pallas import tpu as pltpu
from jax.experimental.pallas import tpu_sc as plsc

_E = 320000
_DIM = 128
_NC = 2
_NS = 16
_NW = _NC * _NS
_BPW = _E // _NW
_CHUNK = 80
_NCHUNK = _BPW // _CHUNK
_NBUF = 5
_L = 16
_VPAD = 3072
_TROWS = _VPAD // _NS


def _edge_embed_body(src_hbm, dst_hbm, emb_hbm, out_hbm,
                     src_v, dst_v, et_v, rows_v, tab_s, gsem, wsem):
    sid = lax.axis_index("s")
    wid = sid * _NC + lax.axis_index("c")
    base = wid * _BPW

    pltpu.sync_copy(emb_hbm.at[pl.ds(sid * _TROWS, _TROWS)],
                    tab_s.at[pl.ds(sid * _TROWS, _TROWS)])

    pltpu.sync_copy(src_hbm.at[pl.ds(base, _BPW)], src_v)
    pltpu.sync_copy(dst_hbm.at[pl.ds(base, _BPW)], dst_v)
    plsc.subcore_barrier()

    def compute_chunk(c):
        for i in range(_CHUNK // _L):
            off = c * _CHUNK + i * _L
            x = src_v[pl.ds(off, _L)]
            y = dst_v[pl.ds(off, _L)]
            a = jnp.abs(x - y) - 1
            et_v[pl.ds(off, _L)] = x * y + ((a * a) >> 2)

    def gather_desc(c, b):
        return pltpu.make_async_copy(
            tab_s.at[et_v.at[pl.ds(c * _CHUNK, _CHUNK)]],
            rows_v.at[b], gsem.at[b])

    def wb_desc(c, b):
        return pltpu.make_async_copy(
            rows_v.at[b],
            out_hbm.at[pl.ds(base + c * _CHUNK, _CHUNK)], wsem.at[b])

    for c in range(_NBUF - 1):
        compute_chunk(c)
        gather_desc(c, c).start()
    gather_desc(0, 0).wait()
    wb_desc(0, 0).start()
    compute_chunk(_NBUF - 1)
    gather_desc(_NBUF - 1, _NBUF - 1).start()

    def block(blk, carry):
        for i in range(_NBUF):
            c = blk * _NBUF + 1 + i
            b = (1 + i) % _NBUF
            bp = i % _NBUF
            gather_desc(c, b).wait()
            wb_desc(c, b).start()
            wb_desc(c - 1, bp).wait()
            compute_chunk(c + _NBUF - 1)
            gather_desc(c + _NBUF - 1, bp).start()
        return carry

    lax.fori_loop(0, (_NCHUNK - _NBUF) // _NBUF, block, 0)

    for i in range(_NBUF - 1):
        c = _NCHUNK - _NBUF + 1 + i
        b = c % _NBUF
        gather_desc(c, b).wait()
        wb_desc(c, b).start()
    for b in range(_NBUF):
        wb_desc(_NCHUNK - _NBUF + b, b).wait()


def kernel(src_node_type, dst_node_type, embedding):
    mesh = plsc.VectorSubcoreMesh(core_axis_name="c", subcore_axis_name="s")
    f = pl.kernel(
        _edge_embed_body,
        out_type=jax.ShapeDtypeStruct((_E, _DIM), jnp.float32),
        mesh=mesh,
        scratch_types=[
            pltpu.VMEM((_BPW,), jnp.int32),
            pltpu.VMEM((_BPW,), jnp.int32),
            pltpu.VMEM((_BPW,), jnp.int32),
            pltpu.VMEM((_NBUF, _CHUNK, _DIM), jnp.float32),
            pltpu.VMEM_SHARED((_VPAD, _DIM), jnp.float32),
            pltpu.SemaphoreType.DMA((_NBUF,)),
            pltpu.SemaphoreType.DMA((_NBUF,)),
        ],
    )
    emb = jnp.pad(embedding, ((0, _VPAD - embedding.shape[0]), (0, 0)))
    return f(src_node_type.astype(jnp.int32),
             dst_node_type.astype(jnp.int32),
             emb)


# speedup vs baseline: 9.0066x; 1.0391x over previous
import jax
import jax.numpy as jnp
from jax import lax
from jax.experimental import pallas as pl
from jax.experimental.pallas import tpu as pltpu
from jax.experimental.pallas import tpu_sc as plsc

_E = 320000
_DIM = 128
_NC = 2
_NS = 16
_NW = _NC * _NS
_BPW = _E // _NW
_CHUNK = 80
_NCHUNK = _BPW // _CHUNK
_NBUF = 5
_L = 16
_VSTAGE = 2432
_TROWS = _VSTAGE // _NS


def _edge_embed_body(src_hbm, dst_hbm, emb_hbm, out_hbm,
                     src_v, dst_v, et_v, rows_v, tab_s, gsem, wsem):
    sid = lax.axis_index("s")
    wid = sid * _NC + lax.axis_index("c")
    base = wid * _BPW

    d_tab = pltpu.make_async_copy(emb_hbm.at[pl.ds(sid * _TROWS, _TROWS)],
                                  tab_s.at[pl.ds(sid * _TROWS, _TROWS)],
                                  gsem.at[0])
    d_src = pltpu.make_async_copy(src_hbm.at[pl.ds(base, _BPW)], src_v,
                                  gsem.at[1])
    d_dst = pltpu.make_async_copy(dst_hbm.at[pl.ds(base, _BPW)], dst_v,
                                  gsem.at[2])
    d_tab.start()
    d_src.start()
    d_dst.start()
    d_tab.wait()
    d_src.wait()
    d_dst.wait()
    plsc.subcore_barrier()

    def compute_chunk(c):
        for i in range(_CHUNK // _L):
            off = c * _CHUNK + i * _L
            x = src_v[pl.ds(off, _L)]
            y = dst_v[pl.ds(off, _L)]
            a = jnp.abs(x - y) - 1
            et_v[pl.ds(off, _L)] = x * y + ((a * a) >> 2)

    def gather_desc(c, b):
        return pltpu.make_async_copy(
            tab_s.at[et_v.at[pl.ds(c * _CHUNK, _CHUNK)]],
            rows_v.at[b], gsem.at[b])

    def wb_desc(c, b):
        return pltpu.make_async_copy(
            rows_v.at[b],
            out_hbm.at[pl.ds(base + c * _CHUNK, _CHUNK)], wsem.at[b])

    for c in range(_NBUF - 1):
        compute_chunk(c)
        gather_desc(c, c).start()
    gather_desc(0, 0).wait()
    wb_desc(0, 0).start()
    compute_chunk(_NBUF - 1)
    gather_desc(_NBUF - 1, _NBUF - 1).start()

    def block(blk, carry):
        for i in range(_NBUF):
            c = blk * _NBUF + 1 + i
            b = (1 + i) % _NBUF
            bp = i % _NBUF
            gather_desc(c, b).wait()
            wb_desc(c, b).start()
            wb_desc(c - 1, bp).wait()
            compute_chunk(c + _NBUF - 1)
            gather_desc(c + _NBUF - 1, bp).start()
        return carry

    lax.fori_loop(0, (_NCHUNK - _NBUF) // _NBUF, block, 0)

    for i in range(_NBUF - 1):
        c = _NCHUNK - _NBUF + 1 + i
        b = c % _NBUF
        gather_desc(c, b).wait()
        wb_desc(c, b).start()
    for b in range(_NBUF):
        wb_desc(_NCHUNK - _NBUF + b, b).wait()


def kernel(src_node_type, dst_node_type, embedding):
    mesh = plsc.VectorSubcoreMesh(core_axis_name="c", subcore_axis_name="s")
    f = pl.kernel(
        _edge_embed_body,
        out_type=jax.ShapeDtypeStruct((_E, _DIM), jnp.float32),
        mesh=mesh,
        scratch_types=[
            pltpu.VMEM((_BPW,), jnp.int32),
            pltpu.VMEM((_BPW,), jnp.int32),
            pltpu.VMEM((_BPW,), jnp.int32),
            pltpu.VMEM((_NBUF, _CHUNK, _DIM), jnp.float32),
            pltpu.VMEM_SHARED((_VSTAGE, _DIM), jnp.float32),
            pltpu.SemaphoreType.DMA((_NBUF,)),
            pltpu.SemaphoreType.DMA((_NBUF,)),
        ],
    )
    return f(src_node_type.astype(jnp.int32),
             dst_node_type.astype(jnp.int32),
             embedding)
